# baseline (speedup 1.0000x reference)
import functools

import jax
import jax.numpy as jnp
from jax import lax
from jax.experimental import pallas as pl
from jax.experimental.pallas import tpu as pltpu
from jax.experimental.pallas import tpu_sc as plsc

NUM_EMBEDDINGS = 4096
FEATURES = 2048
OFFSET = 2

NC = 2
NS = 16
NW = NC * NS

B = 4 * 4096
B_PER_W = B // NW
CHUNK = 16
NCHUNK = B_PER_W // CHUNK
NBUF = 3
NGROUP = NCHUNK // NBUF
NREM = NCHUNK - NGROUP * NBUF


def _emb_kernel(table_hbm, idx_hbm, out_hbm, idx_v, rows_v, in_sem, out_sem):
    wid = lax.axis_index("s") * NC + lax.axis_index("c")
    base = wid * B_PER_W

    pltpu.sync_copy(idx_hbm.at[pl.ds(base, B_PER_W)], idx_v)

    def _add_off(i, carry):
        sl = pl.ds(i * 16, 16)
        idx_v[sl] = idx_v[sl] + OFFSET
        return carry

    lax.fori_loop(0, B_PER_W // 16, _add_off, 0, unroll=4)

    def _gather(c, b):
        return pltpu.make_async_copy(
            table_hbm.at[idx_v.at[pl.ds(c * CHUNK, CHUNK)]],
            rows_v.at[b], in_sem.at[b],
        )

    def _out(c, b):
        return pltpu.make_async_copy(
            rows_v.at[b], out_hbm.at[pl.ds(base + c * CHUNK, CHUNK)],
            out_sem.at[b],
        )

    for b in range(NBUF):
        _gather(b, b).start()

    def _group(g, carry):
        c0 = g * NBUF
        for b in range(NBUF):
            c = c0 + b
            _gather(c, b).wait()
            _out(c, b).start()
        for b in range(NBUF):
            c = c0 + b

            @pl.when(c + NBUF < NCHUNK)
            def _refill():
                _out(c, b).wait()
                _gather(c + NBUF, b).start()

        return carry

    lax.fori_loop(0, NGROUP, _group, 0)

    for r in range(NREM):
        c = NGROUP * NBUF + r
        _gather(c, c % NBUF).wait()
        _out(c, c % NBUF).start()

    for c in range(NCHUNK - NBUF, NCHUNK):
        _out(c, c % NBUF).wait()


@jax.jit
def kernel(inputs, kernel):
    idx_flat = inputs.reshape(-1).astype(jnp.int32)
    call = pl.kernel(
        _emb_kernel,
        out_type=jax.ShapeDtypeStruct((B, FEATURES), jnp.float32),
        mesh=plsc.VectorSubcoreMesh(
            core_axis_name="c", subcore_axis_name="s",
            num_cores=NC, num_subcores=NS,
        ),
        scratch_types=[
            pltpu.VMEM((B_PER_W,), jnp.int32),
            pltpu.VMEM((NBUF, CHUNK, FEATURES), jnp.float32),
            pltpu.SemaphoreType.DMA((NBUF,)),
            pltpu.SemaphoreType.DMA((NBUF,)),
        ],
    )
    out = call(kernel, idx_flat)
    return out.reshape(inputs.shape + (FEATURES,))


# speedup vs baseline: 1.0273x; 1.0273x over previous
import jax
import jax.numpy as jnp
from jax import lax
from jax.experimental import pallas as pl
from jax.experimental.pallas import tpu as pltpu
from jax.experimental.pallas import tpu_sc as plsc

NUM_EMBEDDINGS = 4096
FEATURES = 2048
OFFSET = 2

NC = 2
NS = 16
NW = NC * NS

B = 4 * 4096
B_PER_W = B // NW
CHUNK = 8
NCHUNK = B_PER_W // CHUNK
NBUF = 7
NGROUP = NCHUNK // NBUF
NREM = NCHUNK - NGROUP * NBUF


def _emb_kernel(table_hbm, idx_hbm, out_hbm, idx_v, rows_v, in_sem, out_sem):
    wid = lax.axis_index("s") * NC + lax.axis_index("c")
    base = wid * B_PER_W

    pltpu.sync_copy(idx_hbm.at[pl.ds(base, B_PER_W)], idx_v)

    def _add_off(i, carry):
        sl = pl.ds(i * 16, 16)
        idx_v[sl] = idx_v[sl] + OFFSET
        return carry

    lax.fori_loop(0, B_PER_W // 16, _add_off, 0, unroll=4)

    def _gather(c, b):
        return pltpu.make_async_copy(
            table_hbm.at[idx_v.at[pl.ds(c * CHUNK, CHUNK)]],
            rows_v.at[b], in_sem.at[b],
        )

    def _out(c, b):
        return pltpu.make_async_copy(
            rows_v.at[b], out_hbm.at[pl.ds(base + c * CHUNK, CHUNK)],
            out_sem.at[b],
        )

    for b in range(NBUF):
        _gather(b, b).start()

    def _group(g, carry):
        c0 = g * NBUF
        for b in range(NBUF):
            c = c0 + b
            _gather(c, b).wait()
            _out(c, b).start()
        for b in range(NBUF):
            c = c0 + b

            @pl.when(c + NBUF < NCHUNK)
            def _refill():
                _out(c, b).wait()
                _gather(c + NBUF, b).start()

        return carry

    lax.fori_loop(0, NGROUP, _group, 0)

    for r in range(NREM):
        c = NGROUP * NBUF + r
        _gather(c, c % NBUF).wait()
        _out(c, c % NBUF).start()

    for c in range(NCHUNK - NBUF, NCHUNK):
        _out(c, c % NBUF).wait()


@jax.jit
def kernel(inputs, kernel):
    idx_flat = inputs.reshape(-1).astype(jnp.int32)
    call = pl.kernel(
        _emb_kernel,
        out_type=jax.ShapeDtypeStruct((B, FEATURES), jnp.float32),
        mesh=plsc.VectorSubcoreMesh(
            core_axis_name="c", subcore_axis_name="s",
            num_cores=NC, num_subcores=NS,
        ),
        scratch_types=[
            pltpu.VMEM((B_PER_W,), jnp.int32),
            pltpu.VMEM((NBUF, CHUNK, FEATURES), jnp.float32),
            pltpu.SemaphoreType.DMA((NBUF,)),
            pltpu.SemaphoreType.DMA((NBUF,)),
        ],
    )
    out = call(kernel, idx_flat)
    return out.reshape(inputs.shape + (FEATURES,))
